# baseline (speedup 1.0000x reference)
import functools

import jax
import jax.numpy as jnp
from jax import lax
from jax.experimental import pallas as pl
from jax.experimental.pallas import tpu as pltpu
from jax.experimental.pallas import tpu_sc as plsc

_VOCAB = 100000
_D = 128
_DD = 2 * _D
_B = 4096
_L = 50
_BL = _B * _L

_info = plsc.get_sparse_core_info()
_NC, _NS = _info.num_cores, _info.num_subcores
_NW = _NC * _NS
_PER_W = 2 * _BL // _NW
_C = 128
_N_CH = _PER_W // _C


def _fuse_body(emb_ref, glove_ref, out_ref):
    t = jnp.tanh(emb_ref[:])
    g = glove_ref[:]
    out_ref[:] = jnp.stack((t, g), axis=1).reshape(out_ref.shape)


def _fuse_tables(emb, glove):
    blk = 1000
    return pl.pallas_call(
        _fuse_body,
        grid=(_VOCAB // blk,),
        in_specs=[
            pl.BlockSpec((blk, _D), lambda i: (i, 0)),
            pl.BlockSpec((blk, _D), lambda i: (i, 0)),
        ],
        out_specs=pl.BlockSpec((2 * blk, _D), lambda i: (i, 0)),
        out_shape=jax.ShapeDtypeStruct((2 * _VOCAB, _D), jnp.float32),
    )(emb, glove)


_mesh = plsc.VectorSubcoreMesh(core_axis_name="c", subcore_axis_name="s")


@functools.partial(
    pl.kernel,
    out_type=jax.ShapeDtypeStruct((2 * _BL, _D), jnp.float32),
    mesh=_mesh,
    scratch_types=[
        pltpu.VMEM((_C,), jnp.int32),
        pltpu.VMEM((_C,), jnp.int32),
        pltpu.VMEM((_C, _D), jnp.float32),
        pltpu.VMEM((_C, _D), jnp.float32),
        pltpu.SemaphoreType.DMA,
        pltpu.SemaphoreType.DMA,
    ],
)
def _sc_gather(tbl, ids2, out, idx0, idx1, rows0, rows1, sem0, sem1):
    wid = lax.axis_index("s") * _NC + lax.axis_index("c")
    base = wid * _PER_W

    idx = (idx0, idx1)
    rows = (rows0, rows1)
    sem = (sem0, sem1)

    def start(buf, c):
        pltpu.sync_copy(ids2.at[pl.ds(base + c * _C, _C)], idx[buf])
        pltpu.async_copy(tbl.at[idx[buf]], rows[buf], sem[buf])

    def drain(buf, c):
        pltpu.make_async_copy(tbl.at[idx[buf]], rows[buf], sem[buf]).wait()
        pltpu.sync_copy(rows[buf], out.at[pl.ds(base + c * _C, _C)])

    start(0, 0)

    def pair(g, carry):
        c = 2 * g
        start(1, c + 1)
        drain(0, c)

        @pl.when(c + 2 < _N_CH)
        def _():
            start(0, c + 2)

        drain(1, c + 1)
        return carry

    lax.fori_loop(0, _N_CH // 2, pair, 0)


def kernel(input_ids, emb_table, glove_table):
    ids = input_ids.reshape(-1).astype(jnp.int32)
    ids2 = (ids[:, None] * 2 + jnp.arange(2, dtype=jnp.int32)).reshape(-1)
    tbl = _fuse_tables(emb_table, glove_table)
    out = _sc_gather(tbl, ids2)
    return out.reshape(_B, _L, _DD)


# speedup vs baseline: 1.8548x; 1.8548x over previous
import functools

import jax
import jax.numpy as jnp
from jax import lax
from jax.experimental import pallas as pl
from jax.experimental.pallas import tpu as pltpu
from jax.experimental.pallas import tpu_sc as plsc

_VOCAB = 100000
_D = 128
_DD = 2 * _D
_B = 4096
_L = 50
_BL = _B * _L

_info = plsc.get_sparse_core_info()
_NC, _NS = _info.num_cores, _info.num_subcores
_NW = _NC * _NS
_PER_W = _BL // _NW
_C = 128
_N_CH = _PER_W // _C

_mesh = plsc.VectorSubcoreMesh(core_axis_name="c", subcore_axis_name="s")


@functools.partial(
    pl.kernel,
    out_type=(
        jax.ShapeDtypeStruct((_BL, _D), jnp.float32),
        jax.ShapeDtypeStruct((_BL, _D), jnp.float32),
    ),
    mesh=_mesh,
    scratch_types=[
        pltpu.VMEM((_C,), jnp.int32),
        pltpu.VMEM((_C,), jnp.int32),
        pltpu.VMEM((_C, _D), jnp.float32),
        pltpu.VMEM((_C, _D), jnp.float32),
        pltpu.VMEM((_C, _D), jnp.float32),
        pltpu.VMEM((_C, _D), jnp.float32),
        pltpu.SemaphoreType.DMA,
        pltpu.SemaphoreType.DMA,
        pltpu.SemaphoreType.DMA,
        pltpu.SemaphoreType.DMA,
    ],
)
def _sc_gather(embt, glovet, ids, outE, outG,
               idx0, idx1, rE0, rE1, rG0, rG1, semE0, semE1, semG0, semG1):
    wid = lax.axis_index("s") * _NC + lax.axis_index("c")
    base = wid * _PER_W

    idx = (idx0, idx1)
    rE = (rE0, rE1)
    rG = (rG0, rG1)
    semE = (semE0, semE1)
    semG = (semG0, semG1)

    def start(buf, c):
        pltpu.sync_copy(ids.at[pl.ds(base + c * _C, _C)], idx[buf])
        pltpu.async_copy(embt.at[idx[buf]], rE[buf], semE[buf])
        pltpu.async_copy(glovet.at[idx[buf]], rG[buf], semG[buf])

    def drain(buf, c):
        pltpu.make_async_copy(embt.at[idx[buf]], rE[buf], semE[buf]).wait()
        pltpu.make_async_copy(glovet.at[idx[buf]], rG[buf], semG[buf]).wait()
        pltpu.sync_copy(rE[buf], outE.at[pl.ds(base + c * _C, _C)])
        pltpu.sync_copy(rG[buf], outG.at[pl.ds(base + c * _C, _C)])

    start(0, 0)

    def pair(g, carry):
        c = 2 * g
        start(1, c + 1)
        drain(0, c)

        @pl.when(c + 2 < _N_CH)
        def _():
            start(0, c + 2)

        drain(1, c + 1)
        return carry

    lax.fori_loop(0, _N_CH // 2, pair, 0)


_RB = 64


def _assemble_body(e_ref, g_ref, out_ref):
    out_ref[:, :, :_D] = jnp.tanh(e_ref[:]).reshape(_RB, _L, _D)
    out_ref[:, :, _D:] = g_ref[:].reshape(_RB, _L, _D)


def _assemble(linE, linG):
    return pl.pallas_call(
        _assemble_body,
        grid=(_B // _RB,),
        in_specs=[
            pl.BlockSpec((_RB * _L, _D), lambda i: (i, 0)),
            pl.BlockSpec((_RB * _L, _D), lambda i: (i, 0)),
        ],
        out_specs=pl.BlockSpec((_RB, _L, _DD), lambda i: (i, 0, 0)),
        out_shape=jax.ShapeDtypeStruct((_B, _L, _DD), jnp.float32),
    )(linE, linG)


def kernel(input_ids, emb_table, glove_table):
    ids = input_ids.reshape(-1).astype(jnp.int32)
    outE, outG = _sc_gather(emb_table, glove_table, ids)
    return _assemble(outE, outG)
